# Initial kernel scaffold; baseline (speedup 1.0000x reference)
import jax
import jax.numpy as jnp
from jax.experimental import pallas as pl


def kernel(x, embed_weight, W, b):
    raise NotImplementedError("write your pallas kernel here")


# speedup vs baseline: 17.9045x; 17.9045x over previous
import functools

import jax
import jax.numpy as jnp
from jax import lax
from jax.experimental import pallas as pl
from jax.experimental.pallas import tpu as pltpu
from jax.experimental.pallas import tpu_sc as plsc

_NC = 2
_NS = 16
_LANES = 16
_NW = _NC * _NS


@functools.partial(jax.jit, static_argnums=(3, 4, 5))
def _sc_bag(xr, w0, w1, L, B, NG):
    Vp = w0.shape[0]
    mesh = plsc.VectorSubcoreMesh(core_axis_name="c", subcore_axis_name="s")

    @functools.partial(
        pl.kernel,
        out_type=jax.ShapeDtypeStruct((2 * B,), jnp.float32),
        mesh=mesh,
        scratch_types=[
            pltpu.VMEM((NG * L * _LANES,), jnp.int32),
            pltpu.VMEM((Vp,), jnp.float32),
            pltpu.VMEM((Vp,), jnp.float32),
            pltpu.VMEM((NG * _LANES,), jnp.float32),
            pltpu.VMEM((NG * _LANES,), jnp.float32),
        ],
        compiler_params=pltpu.CompilerParams(needs_layout_passes=False),
    )
    def k(xr_hbm, w0_hbm, w1_hbm, z_hbm, idx_v, w0_v, w1_v, z0_v, z1_v):
        wid = lax.axis_index("s") * _NC + lax.axis_index("c")
        bw = NG * _LANES
        base = wid * bw
        pltpu.sync_copy(xr_hbm.at[wid], idx_v)
        pltpu.sync_copy(w0_hbm, w0_v)
        pltpu.sync_copy(w1_hbm, w1_v)
        for g in range(NG):
            def body(l, carry):
                a0, a1 = carry
                tok = idx_v[pl.ds((g * L + l) * _LANES, _LANES)]
                return (a0 + plsc.load_gather(w0_v, [tok]),
                        a1 + plsc.load_gather(w1_v, [tok]))
            a0, a1 = lax.fori_loop(
                0, L, body,
                (jnp.zeros((_LANES,), jnp.float32),
                 jnp.zeros((_LANES,), jnp.float32)))
            z0_v[pl.ds(g * _LANES, _LANES)] = a0
            z1_v[pl.ds(g * _LANES, _LANES)] = a1
        pltpu.sync_copy(z0_v, z_hbm.at[pl.ds(base, bw)])
        pltpu.sync_copy(z1_v, z_hbm.at[pl.ds(B + base, bw)])

    return k(xr, w0, w1)


def _tail_body(z_ref, b_ref, o_ref):
    s = jax.nn.sigmoid(z_ref[...] + b_ref[...])
    m = jnp.max(s, axis=0, keepdims=True)
    lse = jnp.log(jnp.exp(s[0:1] - m) + jnp.exp(s[1:2] - m)) + m
    o_ref[...] = s - lse


def kernel(x, embed_weight, W, b):
    L, B = x.shape
    OUT, V = W.shape
    del embed_weight
    NG = B // (_NW * _LANES)
    xr = x.reshape(L, _NW, NG, _LANES).transpose(1, 2, 0, 3).reshape(_NW, -1)
    Vp = (V + 127) // 128 * 128
    Wp = jnp.pad(W, ((0, 0), (0, Vp - V)))
    zflat = _sc_bag(xr, Wp[0], Wp[1], L, B, NG)
    z2 = zflat.reshape(2, B)
    out2 = pl.pallas_call(
        _tail_body,
        out_shape=jax.ShapeDtypeStruct((2, B), jnp.float32),
    )(z2, b.reshape(OUT, 1))
    return out2.T
